# baseline (speedup 1.0000x reference)
import functools

import jax
import jax.numpy as jnp
from jax import lax
from jax.experimental import pallas as pl
from jax.experimental.pallas import tpu as pltpu
from jax.experimental.pallas import tpu_sc as plsc

T, H = 2048, 768
E, I, SI = 8, 256, 512
BLK_T = 256
BLK_M = 128
CAP = 5120
NB = CAP // BLK_M
NW = 32
TPW = T // NW
NEG_INF = float("-inf")


def _router_body(x_ref, rw_ref, logits_ref, e1_ref, e2_ref, w1_ref, w2_ref,
                 counts_ref):
    x = x_ref[...]
    rw = rw_ref[...]
    logits = lax.dot_general(x, rw, (((1,), (1,)), ((), ())),
                             preferred_element_type=jnp.float32)
    logits_ref[...] = logits

    lane = jax.lax.broadcasted_iota(jnp.int32, (BLK_T, E), 1)
    m1 = jnp.max(logits, axis=-1, keepdims=True)
    i1 = jnp.min(jnp.where(logits == m1, lane, E), axis=-1, keepdims=True)
    l2 = jnp.where(lane == i1, NEG_INF, logits)
    m2 = jnp.max(l2, axis=-1, keepdims=True)
    i2 = jnp.min(jnp.where(l2 == m2, lane, E), axis=-1, keepdims=True)
    w1 = jax.nn.sigmoid(m1 - m2)

    e1_ref[...] = i1[:, 0]
    e2_ref[...] = i2[:, 0]
    w1_ref[...] = w1[:, 0]
    w2_ref[...] = 1.0 - w1[:, 0]

    lane16 = jax.lax.broadcasted_iota(jnp.int32, (BLK_T, 16), 1)
    oh = (jnp.where(lane16 == i1, 1, 0) + jnp.where(lane16 == i2, 1, 0))
    i = pl.program_id(0)
    c4 = jnp.sum(oh.reshape(BLK_T // TPW, TPW, 16), axis=1)
    counts_ref[pl.ds(i * 4, 4), :] = jnp.concatenate(
        [c4, jnp.zeros((BLK_T // TPW, 112), jnp.int32)], axis=1)


def _router(x2d, router_weight):
    return pl.pallas_call(
        _router_body,
        grid=(T // BLK_T,),
        in_specs=[
            pl.BlockSpec((BLK_T, H), lambda i: (i, 0)),
            pl.BlockSpec((E, H), lambda i: (0, 0)),
        ],
        out_specs=[
            pl.BlockSpec((BLK_T, E), lambda i: (i, 0)),
            pl.BlockSpec((BLK_T,), lambda i: (i,)),
            pl.BlockSpec((BLK_T,), lambda i: (i,)),
            pl.BlockSpec((BLK_T,), lambda i: (i,)),
            pl.BlockSpec((BLK_T,), lambda i: (i,)),
            pl.BlockSpec((NW, 128), lambda i: (0, 0)),
        ],
        out_shape=[
            jax.ShapeDtypeStruct((T, E), jnp.float32),
            jax.ShapeDtypeStruct((T,), jnp.int32),
            jax.ShapeDtypeStruct((T,), jnp.int32),
            jax.ShapeDtypeStruct((T,), jnp.float32),
            jax.ShapeDtypeStruct((T,), jnp.float32),
            jax.ShapeDtypeStruct((NW, 128), jnp.int32),
        ],
    )(x2d, router_weight)


def _shared_body(x_ref, sg_ref, su_ref, sd_ref, seg_ref, shs_ref):
    x = x_ref[...]
    xb = x.astype(jnp.bfloat16)
    sgate = lax.dot_general(xb, sg_ref[...], (((1,), (1,)), ((), ())),
                            preferred_element_type=jnp.float32)
    sup = lax.dot_general(xb, su_ref[...], (((1,), (1,)), ((), ())),
                          preferred_element_type=jnp.float32)
    smid = (jax.nn.silu(sgate) * sup).astype(jnp.bfloat16)
    shared = lax.dot_general(smid, sd_ref[...], (((1,), (1,)), ((), ())),
                             preferred_element_type=jnp.float32)
    g = jax.nn.sigmoid(lax.dot_general(x, seg_ref[...], (((1,), (1,)), ((), ())),
                                       preferred_element_type=jnp.float32))
    shs_ref[...] = g * shared


def _shared(x2d, sg, su, sd, seg):
    return pl.pallas_call(
        _shared_body,
        grid=(T // BLK_T,),
        in_specs=[
            pl.BlockSpec((BLK_T, H), lambda i: (i, 0)),
            pl.BlockSpec((SI, H), lambda i: (0, 0)),
            pl.BlockSpec((SI, H), lambda i: (0, 0)),
            pl.BlockSpec((H, SI), lambda i: (0, 0)),
            pl.BlockSpec((1, H), lambda i: (0, 0)),
        ],
        out_specs=pl.BlockSpec((BLK_T, H), lambda i: (i, 0)),
        out_shape=jax.ShapeDtypeStruct((T, H), jnp.float32),
    )(x2d, sg, su, sd, seg)


@functools.lru_cache(maxsize=None)
def _wide_mesh():
    return plsc.VectorSubcoreMesh(core_axis_name="c", subcore_axis_name="s",
                                  num_cores=2, num_subcores=16)


def _dispatch_body(e1, e2, w1, w2, counts, x_hbm, xg, wslot, inv, blk_e,
                   ev_ref, wp_v, slot_v, cnts_all, sbuf, vecbuf, blk_v,
                   xbuf, semx, sems):
    wid = lax.axis_index("s") * 2 + lax.axis_index("c")
    tb = wid * TPW
    zero16 = jnp.zeros((16,), jnp.int32)
    xcp = pltpu.async_copy(x_hbm.at[pl.ds(tb, TPW)], xbuf, semx)
    pltpu.sync_copy(e1.at[pl.ds(tb, TPW)], ev_ref.at[0])
    pltpu.sync_copy(e2.at[pl.ds(tb, TPW)], ev_ref.at[1])
    pltpu.sync_copy(w1.at[pl.ds(tb, TPW)], wp_v.at[0])
    pltpu.sync_copy(w2.at[pl.ds(tb, TPW)], wp_v.at[1])
    pltpu.sync_copy(counts, cnts_all)

    vecbuf[pl.ds(0, 16)] = zero16

    def prefix_incl(v):
        ps = v
        for sft in (1, 2, 4, 8):
            vecbuf[pl.ds(16, 16)] = ps
            ps = ps + vecbuf[pl.ds(16 - sft, 16)]
        return ps

    tot = zero16
    base = zero16
    for i in range(NW):
        row = cnts_all[i, pl.ds(0, 16)]
        tot = tot + row
        flag = jnp.where(i < wid, 1, 0)
        base = base + row * flag
    padded = ((tot + (BLK_M - 1)) >> 7) << 7
    gs = prefix_incl(padded) - padded
    sv = gs + base
    sbuf[pl.ds(0, 16)] = sv
    sbuf[pl.ds(16, 16)] = gs
    svv = sbuf[pl.ds(0, 16)]
    gsv = sbuf[pl.ds(16, 16)]

    evs = [ev_ref[k, pl.ds(g * 16, 16)] for k in range(2)
           for g in range(TPW // 16)]
    excls = []
    for e in range(E):
        c = zero16
        for ev in evs:
            c = c + jnp.where(ev == e, 1, 0)
        ps = prefix_incl(c)
        excls.append(ps - c)

    curs = [svv[e] + excls[e] for e in range(E)]
    for g, ev in enumerate(evs):
        slot = zero16
        ncurs = []
        for e in range(E):
            m = ev == e
            slot = slot + jnp.where(m, curs[e], zero16)
            ncurs.append(curs[e] + jnp.where(m, 1, 0))
        curs = ncurs
        k, gg = divmod(g, TPW // 16)
        slot_v[k, pl.ds(gg * 16, 16)] = slot

    xcp.wait()
    cps = [
        pltpu.async_copy(xbuf, xg.at[slot_v.at[0]], sems.at[0]),
        pltpu.async_copy(xbuf, xg.at[slot_v.at[1]], sems.at[1]),
        pltpu.async_copy(wp_v.at[0], wslot.at[slot_v.at[0]], sems.at[2]),
        pltpu.async_copy(wp_v.at[1], wslot.at[slot_v.at[1]], sems.at[3]),
    ]
    pltpu.sync_copy(slot_v.at[0], inv.at[pl.ds(tb, TPW)])
    pltpu.sync_copy(slot_v.at[1], inv.at[pl.ds(T + tb, TPW)])
    for cp in cps:
        cp.wait()

    @pl.when(wid == 0)
    def _():
        for r in range(3):
            bvec = (lax.iota(jnp.int32, 16) + r * 16) * BLK_M
            be = zero16
            for e in range(1, E):
                be = be + jnp.where(bvec >= gsv[e], 1, 0)
            blk_v[pl.ds(r * 16, 16)] = be
        pltpu.sync_copy(blk_v, blk_e)


@functools.lru_cache(maxsize=None)
def _make_dispatch():
    return pl.kernel(
        _dispatch_body,
        out_type=[
            jax.ShapeDtypeStruct((CAP, H), jnp.float32),
            jax.ShapeDtypeStruct((CAP,), jnp.float32),
            jax.ShapeDtypeStruct((2 * T,), jnp.int32),
            jax.ShapeDtypeStruct((48,), jnp.int32),
        ],
        mesh=_wide_mesh(),
        scratch_types=[
            pltpu.VMEM((2, TPW), jnp.int32),
            pltpu.VMEM((2, TPW), jnp.float32),
            pltpu.VMEM((2, TPW), jnp.int32),
            pltpu.VMEM((NW, 128), jnp.int32),
            pltpu.VMEM((32,), jnp.int32),
            pltpu.VMEM((32,), jnp.int32),
            pltpu.VMEM((48,), jnp.int32),
            pltpu.VMEM((TPW, H), jnp.float32),
            pltpu.SemaphoreType.DMA,
            pltpu.SemaphoreType.DMA((4,)),
        ],
    )


def _gmm_body(be_ref, xg_ref, gup_ref, down_ref, w_ref, y_ref):
    e = be_ref[pl.program_id(0)]
    x = xg_ref[...].astype(jnp.bfloat16)
    gup = gup_ref[pl.ds(e * 2 * I, 2 * I), :]
    down = down_ref[pl.ds(e * H, H), :]
    gu = lax.dot_general(x, gup, (((1,), (1,)), ((), ())),
                         preferred_element_type=jnp.float32)
    hmid = (jax.nn.silu(gu[:, :I]) * gu[:, I:]).astype(jnp.bfloat16)
    y = lax.dot_general(hmid, down, (((1,), (1,)), ((), ())),
                        preferred_element_type=jnp.float32)
    y_ref[...] = y * w_ref[...]


def _gmm(blk_e, xg, gup, down, wslot2):
    grid_spec = pltpu.PrefetchScalarGridSpec(
        num_scalar_prefetch=1,
        grid=(NB,),
        in_specs=[
            pl.BlockSpec((BLK_M, H), lambda i, be: (i, 0)),
            pl.BlockSpec((E * 2 * I, H), lambda i, be: (0, 0)),
            pl.BlockSpec((E * H, I), lambda i, be: (0, 0)),
            pl.BlockSpec((BLK_M, 1), lambda i, be: (i, 0)),
        ],
        out_specs=pl.BlockSpec((BLK_M, H), lambda i, be: (i, 0)),
    )
    return pl.pallas_call(
        _gmm_body,
        grid_spec=grid_spec,
        out_shape=jax.ShapeDtypeStruct((CAP, H), jnp.float32),
    )(blk_e, xg, gup, down, wslot2)


_CCH = TPW // 2


def _combine_body(y_hbm, inv, shs, out, idx_v, bufa, bufb, bufc, sema, semb):
    wid = lax.axis_index("s") * 2 + lax.axis_index("c")
    tb = wid * TPW
    for c in range(2):
        t0 = tb + c * _CCH
        pltpu.sync_copy(inv.at[pl.ds(t0, _CCH)], idx_v.at[0])
        pltpu.sync_copy(inv.at[pl.ds(T + t0, _CCH)], idx_v.at[1])
        cpa = pltpu.async_copy(y_hbm.at[idx_v.at[0]], bufa, sema)
        cpb = pltpu.async_copy(y_hbm.at[idx_v.at[1]], bufb, semb)
        pltpu.sync_copy(shs.at[pl.ds(t0, _CCH)], bufc)
        cpa.wait()
        cpb.wait()

        def body(j, carry):
            for kk in range(H // 16):
                sl = pl.ds(kk * 16, 16)
                bufc[j, sl] = bufa[j, sl] + bufb[j, sl] + bufc[j, sl]
            return carry
        lax.fori_loop(0, _CCH, body, 0)
        pltpu.sync_copy(bufc, out.at[pl.ds(t0, _CCH)])


@functools.lru_cache(maxsize=None)
def _make_combine():
    return pl.kernel(
        _combine_body,
        out_type=jax.ShapeDtypeStruct((T, H), jnp.float32),
        mesh=_wide_mesh(),
        scratch_types=[
            pltpu.VMEM((2, _CCH), jnp.int32),
            pltpu.VMEM((_CCH, H), jnp.float32),
            pltpu.VMEM((_CCH, H), jnp.float32),
            pltpu.VMEM((_CCH, H), jnp.float32),
            pltpu.SemaphoreType.DMA,
            pltpu.SemaphoreType.DMA,
        ],
    )


def kernel(hidden_states, router_weight, gate_up_proj, down_proj,
           shared_gate_w, shared_up_w, shared_down_w, shared_expert_gate_w):
    b, s, h = hidden_states.shape
    x2d = hidden_states.reshape(-1, h)
    logits, e1, e2, w1, w2, counts = _router(x2d, router_weight)
    xg, wslot, inv, blk_e = _make_dispatch()(e1, e2, w1, w2, counts, x2d)
    shs = _shared(x2d, shared_gate_w.astype(jnp.bfloat16),
                  shared_up_w.astype(jnp.bfloat16),
                  shared_down_w.astype(jnp.bfloat16), shared_expert_gate_w)
    y = _gmm(blk_e, xg, gate_up_proj.astype(jnp.bfloat16).reshape(E * 2 * I, H),
             down_proj.astype(jnp.bfloat16).reshape(E * H, I),
             wslot.reshape(CAP, 1))
    out = _make_combine()(y, inv, shs)
    return out.reshape(b, s, h), logits


# speedup vs baseline: 1.0971x; 1.0971x over previous
import functools

import jax
import jax.numpy as jnp
from jax import lax
from jax.experimental import pallas as pl
from jax.experimental.pallas import tpu as pltpu
from jax.experimental.pallas import tpu_sc as plsc

T, H = 2048, 768
E, I, SI = 8, 256, 512
BLK_T = 256
BLK_M = 256
CAP = 6144
NB = CAP // BLK_M
NW = 32
TPW = T // NW
NEG_INF = float("-inf")


def _router_body(x_ref, rw_ref, logits_ref, e1_ref, e2_ref, w1_ref, w2_ref,
                 counts_ref):
    x = x_ref[...]
    rw = rw_ref[...]
    logits = lax.dot_general(x, rw, (((1,), (1,)), ((), ())),
                             preferred_element_type=jnp.float32)
    logits_ref[...] = logits

    lane = jax.lax.broadcasted_iota(jnp.int32, (BLK_T, E), 1)
    m1 = jnp.max(logits, axis=-1, keepdims=True)
    i1 = jnp.min(jnp.where(logits == m1, lane, E), axis=-1, keepdims=True)
    l2 = jnp.where(lane == i1, NEG_INF, logits)
    m2 = jnp.max(l2, axis=-1, keepdims=True)
    i2 = jnp.min(jnp.where(l2 == m2, lane, E), axis=-1, keepdims=True)
    w1 = jax.nn.sigmoid(m1 - m2)

    e1_ref[...] = i1[:, 0]
    e2_ref[...] = i2[:, 0]
    w1_ref[...] = w1[:, 0]
    w2_ref[...] = 1.0 - w1[:, 0]

    lane16 = jax.lax.broadcasted_iota(jnp.int32, (BLK_T, 16), 1)
    oh = (jnp.where(lane16 == i1, 1, 0) + jnp.where(lane16 == i2, 1, 0))
    i = pl.program_id(0)
    c4 = jnp.sum(oh.reshape(BLK_T // TPW, TPW, 16), axis=1)
    counts_ref[pl.ds(i * 4, 4), :] = jnp.concatenate(
        [c4, jnp.zeros((BLK_T // TPW, 112), jnp.int32)], axis=1)


def _router(x2d, router_weight):
    return pl.pallas_call(
        _router_body,
        grid=(T // BLK_T,),
        in_specs=[
            pl.BlockSpec((BLK_T, H), lambda i: (i, 0)),
            pl.BlockSpec((E, H), lambda i: (0, 0)),
        ],
        out_specs=[
            pl.BlockSpec((BLK_T, E), lambda i: (i, 0)),
            pl.BlockSpec((BLK_T,), lambda i: (i,)),
            pl.BlockSpec((BLK_T,), lambda i: (i,)),
            pl.BlockSpec((BLK_T,), lambda i: (i,)),
            pl.BlockSpec((BLK_T,), lambda i: (i,)),
            pl.BlockSpec((NW, 128), lambda i: (0, 0)),
        ],
        out_shape=[
            jax.ShapeDtypeStruct((T, E), jnp.float32),
            jax.ShapeDtypeStruct((T,), jnp.int32),
            jax.ShapeDtypeStruct((T,), jnp.int32),
            jax.ShapeDtypeStruct((T,), jnp.float32),
            jax.ShapeDtypeStruct((T,), jnp.float32),
            jax.ShapeDtypeStruct((NW, 128), jnp.int32),
        ],
    )(x2d, router_weight)


def _shared_body(x_ref, sg_ref, su_ref, sd_ref, seg_ref, shs_ref):
    x = x_ref[...]
    xb = x.astype(jnp.bfloat16)
    sgate = lax.dot_general(xb, sg_ref[...], (((1,), (1,)), ((), ())),
                            preferred_element_type=jnp.float32)
    sup = lax.dot_general(xb, su_ref[...], (((1,), (1,)), ((), ())),
                          preferred_element_type=jnp.float32)
    smid = (jax.nn.silu(sgate) * sup).astype(jnp.bfloat16)
    shared = lax.dot_general(smid, sd_ref[...], (((1,), (1,)), ((), ())),
                             preferred_element_type=jnp.float32)
    g = jax.nn.sigmoid(lax.dot_general(x, seg_ref[...], (((1,), (1,)), ((), ())),
                                       preferred_element_type=jnp.float32))
    shs_ref[...] = g * shared


def _shared(x2d, sg, su, sd, seg):
    return pl.pallas_call(
        _shared_body,
        grid=(T // BLK_T,),
        in_specs=[
            pl.BlockSpec((BLK_T, H), lambda i: (i, 0)),
            pl.BlockSpec((SI, H), lambda i: (0, 0)),
            pl.BlockSpec((SI, H), lambda i: (0, 0)),
            pl.BlockSpec((H, SI), lambda i: (0, 0)),
            pl.BlockSpec((1, H), lambda i: (0, 0)),
        ],
        out_specs=pl.BlockSpec((BLK_T, H), lambda i: (i, 0)),
        out_shape=jax.ShapeDtypeStruct((T, H), jnp.float32),
    )(x2d, sg, su, sd, seg)


@functools.lru_cache(maxsize=None)
def _wide_mesh():
    return plsc.VectorSubcoreMesh(core_axis_name="c", subcore_axis_name="s",
                                  num_cores=2, num_subcores=16)


def _dispatch_body(e1, e2, w1, w2, counts, x_hbm, xg, wslot, inv, blk_e,
                   ev_ref, wp_v, slot_v, cnts_all, sbuf, vecbuf, blk_v,
                   xbuf, semx, sems):
    wid = lax.axis_index("s") * 2 + lax.axis_index("c")
    tb = wid * TPW
    zero16 = jnp.zeros((16,), jnp.int32)
    xcp = pltpu.async_copy(x_hbm.at[pl.ds(tb, TPW)], xbuf, semx)
    pltpu.sync_copy(e1.at[pl.ds(tb, TPW)], ev_ref.at[0])
    pltpu.sync_copy(e2.at[pl.ds(tb, TPW)], ev_ref.at[1])
    pltpu.sync_copy(w1.at[pl.ds(tb, TPW)], wp_v.at[0])
    pltpu.sync_copy(w2.at[pl.ds(tb, TPW)], wp_v.at[1])
    pltpu.sync_copy(counts, cnts_all)

    vecbuf[pl.ds(0, 16)] = zero16

    def prefix_incl(v):
        ps = v
        for sft in (1, 2, 4, 8):
            vecbuf[pl.ds(16, 16)] = ps
            ps = ps + vecbuf[pl.ds(16 - sft, 16)]
        return ps

    tot = zero16
    base = zero16
    for i in range(NW):
        row = cnts_all[i, pl.ds(0, 16)]
        tot = tot + row
        flag = jnp.where(i < wid, 1, 0)
        base = base + row * flag
    padded = ((tot + (BLK_M - 1)) >> 8) << 8
    gs = prefix_incl(padded) - padded
    sv = gs + base
    sbuf[pl.ds(0, 16)] = sv
    sbuf[pl.ds(16, 16)] = gs
    svv = sbuf[pl.ds(0, 16)]
    gsv = sbuf[pl.ds(16, 16)]

    evs = [ev_ref[k, pl.ds(g * 16, 16)] for k in range(2)
           for g in range(TPW // 16)]
    excls = []
    for e in range(E):
        c = zero16
        for ev in evs:
            c = c + jnp.where(ev == e, 1, 0)
        ps = prefix_incl(c)
        excls.append(ps - c)

    curs = [svv[e] + excls[e] for e in range(E)]
    for g, ev in enumerate(evs):
        slot = zero16
        ncurs = []
        for e in range(E):
            m = ev == e
            slot = slot + jnp.where(m, curs[e], zero16)
            ncurs.append(curs[e] + jnp.where(m, 1, 0))
        curs = ncurs
        k, gg = divmod(g, TPW // 16)
        slot_v[k, pl.ds(gg * 16, 16)] = slot

    xcp.wait()
    cps = [
        pltpu.async_copy(xbuf, xg.at[slot_v.at[0]], sems.at[0]),
        pltpu.async_copy(xbuf, xg.at[slot_v.at[1]], sems.at[1]),
        pltpu.async_copy(wp_v.at[0], wslot.at[slot_v.at[0]], sems.at[2]),
        pltpu.async_copy(wp_v.at[1], wslot.at[slot_v.at[1]], sems.at[3]),
    ]
    pltpu.sync_copy(slot_v.at[0], inv.at[pl.ds(tb, TPW)])
    pltpu.sync_copy(slot_v.at[1], inv.at[pl.ds(T + tb, TPW)])
    for cp in cps:
        cp.wait()

    @pl.when(wid == 0)
    def _():
        for r in range(3):
            bvec = (lax.iota(jnp.int32, 16) + r * 16) * BLK_M
            be = zero16
            for e in range(1, E):
                be = be + jnp.where(bvec >= gsv[e], 1, 0)
            blk_v[pl.ds(r * 16, 16)] = be
        pltpu.sync_copy(blk_v, blk_e)


@functools.lru_cache(maxsize=None)
def _make_dispatch():
    return pl.kernel(
        _dispatch_body,
        out_type=[
            jax.ShapeDtypeStruct((CAP, H), jnp.float32),
            jax.ShapeDtypeStruct((CAP,), jnp.float32),
            jax.ShapeDtypeStruct((2 * T,), jnp.int32),
            jax.ShapeDtypeStruct((48,), jnp.int32),
        ],
        mesh=_wide_mesh(),
        scratch_types=[
            pltpu.VMEM((2, TPW), jnp.int32),
            pltpu.VMEM((2, TPW), jnp.float32),
            pltpu.VMEM((2, TPW), jnp.int32),
            pltpu.VMEM((NW, 128), jnp.int32),
            pltpu.VMEM((32,), jnp.int32),
            pltpu.VMEM((32,), jnp.int32),
            pltpu.VMEM((48,), jnp.int32),
            pltpu.VMEM((TPW, H), jnp.float32),
            pltpu.SemaphoreType.DMA,
            pltpu.SemaphoreType.DMA((4,)),
        ],
    )


def _gmm_body(be_ref, xg_ref, gup_ref, down_ref, w_ref, y_ref):
    e = be_ref[pl.program_id(0)]
    x = xg_ref[...].astype(jnp.bfloat16)
    gup = gup_ref[pl.ds(e * 2 * I, 2 * I), :]
    down = down_ref[pl.ds(e * H, H), :]
    gu = lax.dot_general(x, gup, (((1,), (1,)), ((), ())),
                         preferred_element_type=jnp.float32)
    hmid = (jax.nn.silu(gu[:, :I]) * gu[:, I:]).astype(jnp.bfloat16)
    y = lax.dot_general(hmid, down, (((1,), (1,)), ((), ())),
                        preferred_element_type=jnp.float32)
    y_ref[...] = y * w_ref[...]


def _gmm(blk_e, xg, gup, down, wslot2):
    grid_spec = pltpu.PrefetchScalarGridSpec(
        num_scalar_prefetch=1,
        grid=(NB,),
        in_specs=[
            pl.BlockSpec((BLK_M, H), lambda i, be: (i, 0)),
            pl.BlockSpec((E * 2 * I, H), lambda i, be: (0, 0)),
            pl.BlockSpec((E * H, I), lambda i, be: (0, 0)),
            pl.BlockSpec((BLK_M, 1), lambda i, be: (i, 0)),
        ],
        out_specs=pl.BlockSpec((BLK_M, H), lambda i, be: (i, 0)),
    )
    return pl.pallas_call(
        _gmm_body,
        grid_spec=grid_spec,
        out_shape=jax.ShapeDtypeStruct((CAP, H), jnp.float32),
    )(blk_e, xg, gup, down, wslot2)


_CCH = TPW // 2


def _combine_body(y_hbm, inv, shs, out, idx_v, bufa, bufb, bufc, sema, semb):
    wid = lax.axis_index("s") * 2 + lax.axis_index("c")
    tb = wid * TPW
    for c in range(2):
        t0 = tb + c * _CCH
        pltpu.sync_copy(inv.at[pl.ds(t0, _CCH)], idx_v.at[0])
        pltpu.sync_copy(inv.at[pl.ds(T + t0, _CCH)], idx_v.at[1])
        cpa = pltpu.async_copy(y_hbm.at[idx_v.at[0]], bufa, sema)
        cpb = pltpu.async_copy(y_hbm.at[idx_v.at[1]], bufb, semb)
        pltpu.sync_copy(shs.at[pl.ds(t0, _CCH)], bufc)
        cpa.wait()
        cpb.wait()

        def body(j, carry):
            for kk in range(H // 16):
                sl = pl.ds(kk * 16, 16)
                bufc[j, sl] = bufa[j, sl] + bufb[j, sl] + bufc[j, sl]
            return carry
        lax.fori_loop(0, _CCH, body, 0)
        pltpu.sync_copy(bufc, out.at[pl.ds(t0, _CCH)])


@functools.lru_cache(maxsize=None)
def _make_combine():
    return pl.kernel(
        _combine_body,
        out_type=jax.ShapeDtypeStruct((T, H), jnp.float32),
        mesh=_wide_mesh(),
        scratch_types=[
            pltpu.VMEM((2, _CCH), jnp.int32),
            pltpu.VMEM((_CCH, H), jnp.float32),
            pltpu.VMEM((_CCH, H), jnp.float32),
            pltpu.VMEM((_CCH, H), jnp.float32),
            pltpu.SemaphoreType.DMA,
            pltpu.SemaphoreType.DMA,
        ],
    )


def kernel(hidden_states, router_weight, gate_up_proj, down_proj,
           shared_gate_w, shared_up_w, shared_down_w, shared_expert_gate_w):
    b, s, h = hidden_states.shape
    x2d = hidden_states.reshape(-1, h)
    logits, e1, e2, w1, w2, counts = _router(x2d, router_weight)
    xg, wslot, inv, blk_e = _make_dispatch()(e1, e2, w1, w2, counts, x2d)
    shs = _shared(x2d, shared_gate_w.astype(jnp.bfloat16),
                  shared_up_w.astype(jnp.bfloat16),
                  shared_down_w.astype(jnp.bfloat16), shared_expert_gate_w)
    y = _gmm(blk_e, xg, gate_up_proj.astype(jnp.bfloat16).reshape(E * 2 * I, H),
             down_proj.astype(jnp.bfloat16).reshape(E * H, I),
             wslot.reshape(CAP, 1))
    out = _make_combine()(y, inv, shs)
    return out.reshape(b, s, h), logits
